# baseline (speedup 1.0000x reference)
import functools

import jax
import jax.numpy as jnp
from jax import lax
from jax.experimental import pallas as pl
from jax.experimental.pallas import tpu as pltpu
import jax.experimental.pallas.tpu_sc as plsc

N_TOKENS = 8192
D_MODEL = 768
EXTRA_DIM = 24
HIDDEN = 256
N_EXPERTS = 8
TOP_K = 2
LOSS_COEF = 0.01

TILE = 1024
NTILES = N_TOKENS // TILE

NW = 32
TOK_W = N_TOKENS // NW
L = 16
NGROUPS = TOK_W // L


def _nt_dot(a, b):
    return lax.dot_general(a, b, (((1,), (1,)), ((), ())),
                           preferred_element_type=jnp.float32)


def _dense_body(x_ref, topo_ref, noiset_ref, wcatt_ref, w1x_ref, w1t_ref,
                b1f_ref, w2blkt_ref, b2_ref, noisyt_ref, svectb_ref):
    x = x_ref[...]
    logt = _nt_dot(wcatt_ref[...], x)
    clean = logt[:N_EXPERTS, :]
    raw = logt[N_EXPERTS:, :]
    sp = jnp.maximum(raw, 0.0) + jnp.log1p(jnp.exp(-jnp.abs(raw)))
    noisyt_ref[...] = clean + noiset_ref[...] * (sp + 1e-2)

    h = (jnp.dot(x.astype(jnp.bfloat16), w1x_ref[...],
                 preferred_element_type=jnp.float32)
         + jnp.dot(topo_ref[...], w1t_ref[...],
                   preferred_element_type=jnp.float32)
         + b1f_ref[...])
    h = jnp.maximum(h, 0.0).astype(jnp.bfloat16)
    svectb_ref[...] = _nt_dot(w2blkt_ref[...], h) + b2_ref[...]


def _router_body(noisyt_hbm, svectb_hbm, y_hbm, stats_hbm,
                 nbuf, sbuf, ybuf, statbuf):
    wid = lax.axis_index("s") * 2 + lax.axis_index("c")
    base = wid * TOK_W
    pltpu.sync_copy(noisyt_hbm.at[:, pl.ds(base, TOK_W)], nbuf)
    pltpu.sync_copy(svectb_hbm.at[:, pl.ds(base, TOK_W)], sbuf)

    zero = jnp.zeros((L,), jnp.float32)
    imp = [zero for _ in range(N_EXPERTS)]
    load = [zero for _ in range(N_EXPERTS)]
    neg_inf = jnp.full((L,), -jnp.inf, jnp.float32)

    for g in range(NGROUPS):
        sl = pl.ds(g * L, L)
        nv = [nbuf[e, sl] for e in range(N_EXPERTS)]
        sv = [sbuf[e, sl] for e in range(N_EXPERTS)]
        m1 = nv[0]
        i1 = jnp.zeros((L,), jnp.int32)
        for e in range(1, N_EXPERTS):
            c = nv[e] > m1
            m1 = jnp.where(c, nv[e], m1)
            i1 = jnp.where(c, jnp.full((L,), e, jnp.int32), i1)
        m2 = neg_inf
        i2 = jnp.zeros((L,), jnp.int32)
        for e in range(N_EXPERTS):
            v = jnp.where(i1 == e, neg_inf, nv[e])
            c = v > m2
            m2 = jnp.where(c, v, m2)
            i2 = jnp.where(c, jnp.full((L,), e, jnp.int32), i2)
        t = jnp.exp(m2 - m1)
        denom = 1.0 + t
        g1 = 1.0 / denom
        g2 = t / denom
        one = jnp.ones((L,), jnp.float32)
        yv = zero
        for e in range(N_EXPERTS):
            s1 = i1 == e
            s2 = i2 == e
            ge = jnp.where(s1, g1, zero) + jnp.where(s2, g2, zero)
            yv = yv + ge * sv[e]
            imp[e] = imp[e] + ge
            load[e] = (load[e] + jnp.where(s1, one, zero)
                       + jnp.where(s2 & (g2 > 0.0), one, zero))
        ybuf[sl] = yv

    pltpu.sync_copy(ybuf, y_hbm.at[pl.ds(base, TOK_W)])
    for e in range(N_EXPERTS):
        statbuf[pl.ds(e * L, L)] = imp[e]
        statbuf[pl.ds((N_EXPERTS + e) * L, L)] = load[e]
    pltpu.sync_copy(statbuf, stats_hbm.at[wid])


def _cv_sq(v):
    eps = 1e-10
    m = jnp.mean(v)
    var = jnp.mean((v - m) ** 2)
    return var / (m * m + eps)


def _aux_body(stats_ref, aux_ref):
    st = stats_ref[...]
    imp_s = [jnp.sum(st[:, e * L:(e + 1) * L]) for e in range(N_EXPERTS)]
    load_s = [jnp.sum(st[:, (N_EXPERTS + e) * L:(N_EXPERTS + e + 1) * L])
              for e in range(N_EXPERTS)]

    def cv2(vals):
        eps = 1e-10
        m = sum(vals) / N_EXPERTS
        var = sum((v - m) ** 2 for v in vals) / N_EXPERTS
        return var / (m * m + eps)

    aux = LOSS_COEF * (cv2(imp_s) + cv2(load_s))
    aux_ref[...] = aux.reshape(1, 1)


@functools.partial(jax.jit, static_argnames=("interpret",))
def kernel(x, node_topo_features, noise, w_gate, w_noise, W1, b1, W2, b2,
           interpret=False):
    wcatt = jnp.concatenate([w_gate, w_noise], axis=1).T
    noiset = noise.T
    w1r = jnp.transpose(W1, (1, 0, 2)).reshape(D_MODEL + EXTRA_DIM,
                                               N_EXPERTS * HIDDEN)
    w1x = w1r[:D_MODEL].astype(jnp.bfloat16)
    w1t = w1r[D_MODEL:].astype(jnp.bfloat16)
    topo_bf = node_topo_features.astype(jnp.bfloat16)
    b1f = b1.reshape(1, N_EXPERTS * HIDDEN)
    eye = jnp.eye(N_EXPERTS, dtype=jnp.float32)
    w2blkt = (eye[:, :, None] * W2[None, :, :, 0]).reshape(
        N_EXPERTS, N_EXPERTS * HIDDEN).astype(jnp.bfloat16)
    b2c = b2

    noisyt, svectb = pl.pallas_call(
        _dense_body,
        grid=(NTILES,),
        in_specs=[
            pl.BlockSpec((TILE, D_MODEL), lambda i: (i, 0)),
            pl.BlockSpec((TILE, EXTRA_DIM), lambda i: (i, 0)),
            pl.BlockSpec((N_EXPERTS, TILE), lambda i: (0, i)),
            pl.BlockSpec((2 * N_EXPERTS, D_MODEL), lambda i: (0, 0)),
            pl.BlockSpec((D_MODEL, N_EXPERTS * HIDDEN), lambda i: (0, 0)),
            pl.BlockSpec((EXTRA_DIM, N_EXPERTS * HIDDEN), lambda i: (0, 0)),
            pl.BlockSpec((1, N_EXPERTS * HIDDEN), lambda i: (0, 0)),
            pl.BlockSpec((N_EXPERTS, N_EXPERTS * HIDDEN), lambda i: (0, 0)),
            pl.BlockSpec((N_EXPERTS, 1), lambda i: (0, 0)),
        ],
        out_specs=[
            pl.BlockSpec((N_EXPERTS, TILE), lambda i: (0, i)),
            pl.BlockSpec((N_EXPERTS, TILE), lambda i: (0, i)),
        ],
        out_shape=[
            jax.ShapeDtypeStruct((N_EXPERTS, N_TOKENS), jnp.float32),
            jax.ShapeDtypeStruct((N_EXPERTS, N_TOKENS), jnp.float32),
        ],
        interpret=interpret,
    )(x, topo_bf, noiset, wcatt, w1x, w1t, b1f, w2blkt, b2c)

    router = pl.kernel(
        _router_body,
        out_type=[
            jax.ShapeDtypeStruct((N_TOKENS,), jnp.float32),
            jax.ShapeDtypeStruct((NW, 2 * N_EXPERTS * L), jnp.float32),
        ],
        mesh=plsc.VectorSubcoreMesh(core_axis_name="c",
                                    subcore_axis_name="s"),
        scratch_types=[
            pltpu.VMEM((N_EXPERTS, TOK_W), jnp.float32),
            pltpu.VMEM((N_EXPERTS, TOK_W), jnp.float32),
            pltpu.VMEM((TOK_W,), jnp.float32),
            pltpu.VMEM((2 * N_EXPERTS * L,), jnp.float32),
        ],
    )
    y, stats = router(noisyt, svectb)

    aux = pl.pallas_call(
        _aux_body,
        out_shape=jax.ShapeDtypeStruct((1, 1), jnp.float32),
        interpret=interpret,
    )(stats)
    return y, aux[0, 0]


# speedup vs baseline: 1.0038x; 1.0038x over previous
import functools

import jax
import jax.numpy as jnp
from jax import lax
from jax.experimental import pallas as pl
from jax.experimental.pallas import tpu as pltpu
import jax.experimental.pallas.tpu_sc as plsc

N_TOKENS = 8192
D_MODEL = 768
EXTRA_DIM = 24
HIDDEN = 256
N_EXPERTS = 8
TOP_K = 2
LOSS_COEF = 0.01

TILE = 1024
NTILES = N_TOKENS // TILE

NW = 32
TOK_W = N_TOKENS // NW
L = 16
NGROUPS = TOK_W // L


def _nt_dot(a, b):
    return lax.dot_general(a, b, (((1,), (1,)), ((), ())),
                           preferred_element_type=jnp.float32)


def _cv_sq(v):
    eps = 1e-10
    m = jnp.mean(v)
    var = jnp.mean((v - m) ** 2)
    return var / (m * m + eps)


def _dense_body(x_ref, topo_ref, noiset_ref, wcatt_ref, w1x_ref, w1t_ref,
                b1f_ref, w2blkt_ref, b2_ref,
                noisyt_ref, svectb_ref, imp_ref, load_ref, aux_ref):
    i = pl.program_id(0)
    x = x_ref[...]
    logt = _nt_dot(wcatt_ref[...], x)
    clean = logt[:N_EXPERTS, :]
    raw = logt[N_EXPERTS:, :]
    sp = jnp.maximum(raw, 0.0) + jnp.log1p(jnp.exp(-jnp.abs(raw)))
    noisy = clean + noiset_ref[...] * (sp + 1e-2)
    noisyt_ref[...] = noisy

    sio = lax.broadcasted_iota(jnp.int32, (N_EXPERTS, TILE), 0)
    v1 = jnp.max(noisy, axis=0, keepdims=True)
    idx1 = jnp.min(jnp.where(noisy >= v1, sio, N_EXPERTS), axis=0,
                   keepdims=True)
    sel1 = sio == idx1
    noisy2 = jnp.where(sel1, -jnp.inf, noisy)
    v2 = jnp.max(noisy2, axis=0, keepdims=True)
    idx2 = jnp.min(jnp.where(noisy2 >= v2, sio, N_EXPERTS), axis=0,
                   keepdims=True)
    sel2 = sio == idx2
    t = jnp.exp(v2 - v1)
    g1 = 1.0 / (1.0 + t)
    g2 = t / (1.0 + t)
    gatest = jnp.where(sel1, g1, 0.0) + jnp.where(sel2, g2, 0.0)

    imp_part = jnp.sum(gatest, axis=1, keepdims=True)
    load_part = jnp.sum((gatest > 0.0).astype(jnp.float32), axis=1,
                        keepdims=True)

    @pl.when(i == 0)
    def _():
        imp_ref[...] = imp_part
        load_ref[...] = load_part

    @pl.when(i > 0)
    def _():
        imp_ref[...] += imp_part
        load_ref[...] += load_part

    h = (jnp.dot(x.astype(jnp.bfloat16), w1x_ref[...],
                 preferred_element_type=jnp.float32)
         + jnp.dot(topo_ref[...], w1t_ref[...],
                   preferred_element_type=jnp.float32)
         + b1f_ref[...])
    h = jnp.maximum(h, 0.0).astype(jnp.bfloat16)
    svectb_ref[...] = _nt_dot(w2blkt_ref[...], h) + b2_ref[...]

    @pl.when(i == NTILES - 1)
    def _():
        aux = LOSS_COEF * (_cv_sq(imp_ref[...]) + _cv_sq(load_ref[...]))
        aux_ref[...] = aux.reshape(1, 1)


def _router_body(noisyt_hbm, svectb_hbm, y_hbm, nbuf, sbuf, ybuf):
    wid = lax.axis_index("s") * 2 + lax.axis_index("c")
    base = wid * TOK_W
    pltpu.sync_copy(noisyt_hbm.at[:, pl.ds(base, TOK_W)], nbuf)
    pltpu.sync_copy(svectb_hbm.at[:, pl.ds(base, TOK_W)], sbuf)

    zero = jnp.zeros((L,), jnp.float32)
    neg_inf = jnp.full((L,), -jnp.inf, jnp.float32)

    for g in range(NGROUPS):
        sl = pl.ds(g * L, L)
        nv = [nbuf[e, sl] for e in range(N_EXPERTS)]
        sv = [sbuf[e, sl] for e in range(N_EXPERTS)]
        m1 = nv[0]
        i1 = jnp.zeros((L,), jnp.int32)
        for e in range(1, N_EXPERTS):
            c = nv[e] > m1
            m1 = jnp.where(c, nv[e], m1)
            i1 = jnp.where(c, jnp.full((L,), e, jnp.int32), i1)
        m2 = neg_inf
        i2 = jnp.zeros((L,), jnp.int32)
        for e in range(N_EXPERTS):
            v = jnp.where(i1 == e, neg_inf, nv[e])
            c = v > m2
            m2 = jnp.where(c, v, m2)
            i2 = jnp.where(c, jnp.full((L,), e, jnp.int32), i2)
        t = jnp.exp(m2 - m1)
        denom = 1.0 + t
        g1 = 1.0 / denom
        g2 = t / denom
        yv = zero
        for e in range(N_EXPERTS):
            ge = (jnp.where(i1 == e, g1, zero)
                  + jnp.where(i2 == e, g2, zero))
            yv = yv + ge * sv[e]
        ybuf[sl] = yv

    pltpu.sync_copy(ybuf, y_hbm.at[pl.ds(base, TOK_W)])


@functools.partial(jax.jit, static_argnames=("interpret",))
def kernel(x, node_topo_features, noise, w_gate, w_noise, W1, b1, W2, b2,
           interpret=False):
    wcatt = jnp.concatenate([w_gate, w_noise], axis=1).T
    noiset = noise.T
    w1r = jnp.transpose(W1, (1, 0, 2)).reshape(D_MODEL + EXTRA_DIM,
                                               N_EXPERTS * HIDDEN)
    w1x = w1r[:D_MODEL].astype(jnp.bfloat16)
    w1t = w1r[D_MODEL:].astype(jnp.bfloat16)
    topo_bf = node_topo_features.astype(jnp.bfloat16)
    b1f = b1.reshape(1, N_EXPERTS * HIDDEN)
    eye = jnp.eye(N_EXPERTS, dtype=jnp.float32)
    w2blkt = (eye[:, :, None] * W2[None, :, :, 0]).reshape(
        N_EXPERTS, N_EXPERTS * HIDDEN).astype(jnp.bfloat16)
    b2c = b2

    noisyt, svectb, _imp, _load, aux = pl.pallas_call(
        _dense_body,
        grid=(NTILES,),
        in_specs=[
            pl.BlockSpec((TILE, D_MODEL), lambda i: (i, 0)),
            pl.BlockSpec((TILE, EXTRA_DIM), lambda i: (i, 0)),
            pl.BlockSpec((N_EXPERTS, TILE), lambda i: (0, i)),
            pl.BlockSpec((2 * N_EXPERTS, D_MODEL), lambda i: (0, 0)),
            pl.BlockSpec((D_MODEL, N_EXPERTS * HIDDEN), lambda i: (0, 0)),
            pl.BlockSpec((EXTRA_DIM, N_EXPERTS * HIDDEN), lambda i: (0, 0)),
            pl.BlockSpec((1, N_EXPERTS * HIDDEN), lambda i: (0, 0)),
            pl.BlockSpec((N_EXPERTS, N_EXPERTS * HIDDEN), lambda i: (0, 0)),
            pl.BlockSpec((N_EXPERTS, 1), lambda i: (0, 0)),
        ],
        out_specs=[
            pl.BlockSpec((N_EXPERTS, TILE), lambda i: (0, i)),
            pl.BlockSpec((N_EXPERTS, TILE), lambda i: (0, i)),
            pl.BlockSpec((N_EXPERTS, 1), lambda i: (0, 0)),
            pl.BlockSpec((N_EXPERTS, 1), lambda i: (0, 0)),
            pl.BlockSpec((1, 1), lambda i: (0, 0)),
        ],
        out_shape=[
            jax.ShapeDtypeStruct((N_EXPERTS, N_TOKENS), jnp.float32),
            jax.ShapeDtypeStruct((N_EXPERTS, N_TOKENS), jnp.float32),
            jax.ShapeDtypeStruct((N_EXPERTS, 1), jnp.float32),
            jax.ShapeDtypeStruct((N_EXPERTS, 1), jnp.float32),
            jax.ShapeDtypeStruct((1, 1), jnp.float32),
        ],
        interpret=interpret,
    )(x, topo_bf, noiset, wcatt, w1x, w1t, b1f, w2blkt, b2c)

    router = pl.kernel(
        _router_body,
        out_type=jax.ShapeDtypeStruct((N_TOKENS,), jnp.float32),
        mesh=plsc.VectorSubcoreMesh(core_axis_name="c",
                                    subcore_axis_name="s"),
        scratch_types=[
            pltpu.VMEM((N_EXPERTS, TOK_W), jnp.float32),
            pltpu.VMEM((N_EXPERTS, TOK_W), jnp.float32),
            pltpu.VMEM((TOK_W,), jnp.float32),
        ],
    )
    y = router(noisyt, svectb)
    return y, aux[0, 0]


# speedup vs baseline: 1.0054x; 1.0016x over previous
import functools

import jax
import jax.numpy as jnp
from jax import lax
from jax.experimental import pallas as pl
from jax.experimental.pallas import tpu as pltpu
import jax.experimental.pallas.tpu_sc as plsc

N_TOKENS = 8192
D_MODEL = 768
EXTRA_DIM = 24
HIDDEN = 256
N_EXPERTS = 8
TOP_K = 2
LOSS_COEF = 0.01

TILE = 2048
NTILES = N_TOKENS // TILE

NW = 32
TOK_W = N_TOKENS // NW
L = 16
NGROUPS = TOK_W // L


def _nt_dot(a, b):
    return lax.dot_general(a, b, (((1,), (1,)), ((), ())),
                           preferred_element_type=jnp.float32)


def _cv_sq(v):
    eps = 1e-10
    m = jnp.mean(v)
    var = jnp.mean((v - m) ** 2)
    return var / (m * m + eps)


def _dense_body(x_ref, topo_ref, noiset_ref, wcatt_ref, w1x_ref, w1t_ref,
                b1f_ref, w2blkt_ref, b2_ref,
                noisyt_ref, svectb_ref, imp_ref, load_ref, aux_ref):
    i = pl.program_id(0)
    x = x_ref[...]
    logt = _nt_dot(wcatt_ref[...], x)
    clean = logt[:N_EXPERTS, :]
    raw = logt[N_EXPERTS:, :]
    sp = jnp.maximum(raw, 0.0) + jnp.log1p(jnp.exp(-jnp.abs(raw)))
    noisy = clean + noiset_ref[...] * (sp + 1e-2)
    noisyt_ref[...] = noisy

    sio = lax.broadcasted_iota(jnp.int32, (N_EXPERTS, TILE), 0)
    v1 = jnp.max(noisy, axis=0, keepdims=True)
    idx1 = jnp.min(jnp.where(noisy >= v1, sio, N_EXPERTS), axis=0,
                   keepdims=True)
    sel1 = sio == idx1
    noisy2 = jnp.where(sel1, -jnp.inf, noisy)
    v2 = jnp.max(noisy2, axis=0, keepdims=True)
    idx2 = jnp.min(jnp.where(noisy2 >= v2, sio, N_EXPERTS), axis=0,
                   keepdims=True)
    sel2 = sio == idx2
    t = jnp.exp(v2 - v1)
    g1 = 1.0 / (1.0 + t)
    g2 = t / (1.0 + t)
    gatest = jnp.where(sel1, g1, 0.0) + jnp.where(sel2, g2, 0.0)

    imp_part = jnp.sum(gatest, axis=1, keepdims=True)
    load_part = jnp.sum((gatest > 0.0).astype(jnp.float32), axis=1,
                        keepdims=True)

    @pl.when(i == 0)
    def _():
        imp_ref[...] = imp_part
        load_ref[...] = load_part

    @pl.when(i > 0)
    def _():
        imp_ref[...] += imp_part
        load_ref[...] += load_part

    h = (jnp.dot(x.astype(jnp.bfloat16), w1x_ref[...],
                 preferred_element_type=jnp.float32)
         + jnp.dot(topo_ref[...], w1t_ref[...],
                   preferred_element_type=jnp.float32)
         + b1f_ref[...])
    h = jnp.maximum(h, 0.0).astype(jnp.bfloat16)
    svectb_ref[...] = _nt_dot(w2blkt_ref[...], h) + b2_ref[...]

    @pl.when(i == NTILES - 1)
    def _():
        aux = LOSS_COEF * (_cv_sq(imp_ref[...]) + _cv_sq(load_ref[...]))
        aux_ref[...] = aux.reshape(1, 1)


def _router_body(noisyt_hbm, svectb_hbm, y_hbm, nbuf, sbuf, ybuf):
    wid = lax.axis_index("s") * 2 + lax.axis_index("c")
    base = wid * TOK_W
    pltpu.sync_copy(noisyt_hbm.at[:, pl.ds(base, TOK_W)], nbuf)
    pltpu.sync_copy(svectb_hbm.at[:, pl.ds(base, TOK_W)], sbuf)

    zero = jnp.zeros((L,), jnp.float32)
    neg_inf = jnp.full((L,), -jnp.inf, jnp.float32)

    for g in range(NGROUPS):
        sl = pl.ds(g * L, L)
        nv = [nbuf[e, sl] for e in range(N_EXPERTS)]
        sv = [sbuf[e, sl] for e in range(N_EXPERTS)]
        m1 = nv[0]
        i1 = jnp.zeros((L,), jnp.int32)
        for e in range(1, N_EXPERTS):
            c = nv[e] > m1
            m1 = jnp.where(c, nv[e], m1)
            i1 = jnp.where(c, jnp.full((L,), e, jnp.int32), i1)
        m2 = neg_inf
        i2 = jnp.zeros((L,), jnp.int32)
        for e in range(N_EXPERTS):
            v = jnp.where(i1 == e, neg_inf, nv[e])
            c = v > m2
            m2 = jnp.where(c, v, m2)
            i2 = jnp.where(c, jnp.full((L,), e, jnp.int32), i2)
        t = jnp.exp(m2 - m1)
        denom = 1.0 + t
        g1 = 1.0 / denom
        g2 = t / denom
        yv = zero
        for e in range(N_EXPERTS):
            ge = (jnp.where(i1 == e, g1, zero)
                  + jnp.where(i2 == e, g2, zero))
            yv = yv + ge * sv[e]
        ybuf[sl] = yv

    pltpu.sync_copy(ybuf, y_hbm.at[pl.ds(base, TOK_W)])


@functools.partial(jax.jit, static_argnames=("interpret",))
def kernel(x, node_topo_features, noise, w_gate, w_noise, W1, b1, W2, b2,
           interpret=False):
    wcatt = jnp.concatenate([w_gate, w_noise], axis=1).T
    noiset = noise.T
    w1r = jnp.transpose(W1, (1, 0, 2)).reshape(D_MODEL + EXTRA_DIM,
                                               N_EXPERTS * HIDDEN)
    w1x = w1r[:D_MODEL].astype(jnp.bfloat16)
    w1t = w1r[D_MODEL:].astype(jnp.bfloat16)
    topo_bf = node_topo_features.astype(jnp.bfloat16)
    b1f = b1.reshape(1, N_EXPERTS * HIDDEN)
    eye = jnp.eye(N_EXPERTS, dtype=jnp.float32)
    w2blkt = (eye[:, :, None] * W2[None, :, :, 0]).reshape(
        N_EXPERTS, N_EXPERTS * HIDDEN).astype(jnp.bfloat16)
    b2c = b2

    noisyt, svectb, _imp, _load, aux = pl.pallas_call(
        _dense_body,
        grid=(NTILES,),
        in_specs=[
            pl.BlockSpec((TILE, D_MODEL), lambda i: (i, 0)),
            pl.BlockSpec((TILE, EXTRA_DIM), lambda i: (i, 0)),
            pl.BlockSpec((N_EXPERTS, TILE), lambda i: (0, i)),
            pl.BlockSpec((2 * N_EXPERTS, D_MODEL), lambda i: (0, 0)),
            pl.BlockSpec((D_MODEL, N_EXPERTS * HIDDEN), lambda i: (0, 0)),
            pl.BlockSpec((EXTRA_DIM, N_EXPERTS * HIDDEN), lambda i: (0, 0)),
            pl.BlockSpec((1, N_EXPERTS * HIDDEN), lambda i: (0, 0)),
            pl.BlockSpec((N_EXPERTS, N_EXPERTS * HIDDEN), lambda i: (0, 0)),
            pl.BlockSpec((N_EXPERTS, 1), lambda i: (0, 0)),
        ],
        out_specs=[
            pl.BlockSpec((N_EXPERTS, TILE), lambda i: (0, i)),
            pl.BlockSpec((N_EXPERTS, TILE), lambda i: (0, i)),
            pl.BlockSpec((N_EXPERTS, 1), lambda i: (0, 0)),
            pl.BlockSpec((N_EXPERTS, 1), lambda i: (0, 0)),
            pl.BlockSpec((1, 1), lambda i: (0, 0)),
        ],
        out_shape=[
            jax.ShapeDtypeStruct((N_EXPERTS, N_TOKENS), jnp.float32),
            jax.ShapeDtypeStruct((N_EXPERTS, N_TOKENS), jnp.float32),
            jax.ShapeDtypeStruct((N_EXPERTS, 1), jnp.float32),
            jax.ShapeDtypeStruct((N_EXPERTS, 1), jnp.float32),
            jax.ShapeDtypeStruct((1, 1), jnp.float32),
        ],
        interpret=interpret,
    )(x, topo_bf, noiset, wcatt, w1x, w1t, b1f, w2blkt, b2c)

    router = pl.kernel(
        _router_body,
        out_type=jax.ShapeDtypeStruct((N_TOKENS,), jnp.float32),
        mesh=plsc.VectorSubcoreMesh(core_axis_name="c",
                                    subcore_axis_name="s"),
        scratch_types=[
            pltpu.VMEM((N_EXPERTS, TOK_W), jnp.float32),
            pltpu.VMEM((N_EXPERTS, TOK_W), jnp.float32),
            pltpu.VMEM((TOK_W,), jnp.float32),
        ],
    )
    y = router(noisyt, svectb)
    return y, aux[0, 0]
